# baseline (speedup 1.0000x reference)
import functools

import jax
import jax.numpy as jnp
from jax import lax
from jax.experimental import pallas as pl
from jax.experimental.pallas import tpu as pltpu
from jax.experimental.pallas import tpu_sc as plsc

N = 10000
D = 128
R = 8
B = 4
E = 320000

NC, NS, L = 2, 16, 16
NW = NC * NS
CH = 64
NCHUNK = 160
EPW = NCHUNK * CH
EP = NW * EPW
PAD = EP - E
NACC = 10240
RPT = NACC // NS
TBL = N * (R + 1)
BN = 2000
BF = 1024


def _xf_body(wc, bases, lw, x, o):
    xb = x[...]
    cols = []
    for r in range(R):
        w = wc[r, 0] * bases[0]
        for b in range(1, B):
            w = w + wc[r, b] * bases[b]
        cols.append(w)
    cols.append(lw[...])
    wall = jnp.concatenate(cols, axis=1)
    full = jnp.dot(xb, wall, preferred_element_type=jnp.float32)
    for r in range(R + 1):
        o[r] = full[:, r * D:(r + 1) * D]


def _xform(x, wcomp, bases, loopw):
    return pl.pallas_call(
        _xf_body,
        grid=(N // BN,),
        in_specs=[
            pl.BlockSpec(memory_space=pltpu.SMEM),
            pl.BlockSpec((B, D, D), lambda i: (0, 0, 0)),
            pl.BlockSpec((D, D), lambda i: (0, 0)),
            pl.BlockSpec((BN, D), lambda i: (i, 0)),
        ],
        out_specs=pl.BlockSpec((R + 1, BN, D), lambda i: (0, i, 0)),
        out_shape=jax.ShapeDtypeStruct((R + 1, N, D), jnp.float32),
    )(wcomp, bases, loopw, x)


GCH = 32
NGRP = NCHUNK // GCH
NBUF = 4
GRP_C0 = 5
GRP_C1 = 5
CPS = GCH * (GRP_C0 + GRP_C1)


def _sc_body(table, srch, eth, dsth, aggo,
             srcg, etg, gidx, dstg, rows0, rows1, rows2, rows3,
             acc, sem0, sem1, sem2, sem3):
    c = lax.axis_index("c")
    s = lax.axis_index("s")
    rows = (rows0, rows1, rows2, rows3)
    sems = (sem0, sem1, sem2, sem3)

    zero16 = jnp.zeros((L,), jnp.float32)

    def zrow(i, _):
        for k in range(D // L):
            rows0[i, pl.ds(k * L, L)] = zero16
        return 0
    lax.fori_loop(0, CH, zrow, 0)
    for m in range(RPT // CH):
        pltpu.sync_copy(rows0, acc.at[pl.ds(s * RPT + m * CH, CH)])

    plsc.subcore_barrier()

    base = s * CPS + c * (GCH * GRP_C0)
    ngrp = jnp.where(c == 0, GRP_C0, GRP_C1)

    def group(g, _):
        row0 = base + g * GCH
        pltpu.sync_copy(srch.at[pl.ds(row0, GCH)], srcg)
        pltpu.sync_copy(eth.at[pl.ds(row0, GCH)], etg)
        pltpu.sync_copy(dsth.at[pl.ds(row0, GCH)], dstg)

        def gix(j, _):
            for k in range(CH // L):
                sl = pl.ds(k * L, L)
                gidx[j, sl] = etg[j, sl] * N + srcg[j, sl]
            return 0
        lax.fori_loop(0, GCH, gix, 0)

        for b in range(NBUF - 1):
            pltpu.async_copy(table.at[gidx.at[b]], rows[b], sems[b])

        def quad(q, _):
            for b in range(NBUF):
                j = NBUF * q + b
                pltpu.make_async_copy(table.at[gidx.at[j]],
                                      rows[b], sems[b]).wait()

                @pl.when(j + NBUF - 1 < GCH)
                def _():
                    bn = (b + NBUF - 1) % NBUF
                    pltpu.async_copy(table.at[gidx.at[j + NBUF - 1]],
                                     rows[bn], sems[bn])

                pltpu.sync_copy(rows[b], acc.at[dstg.at[j]], add=True)
            return 0
        lax.fori_loop(0, GCH // NBUF, quad, 0)
        return 0
    lax.fori_loop(0, ngrp, group, 0)

    plsc.subcore_barrier()

    for m in range(RPT // CH):
        off = s * RPT + m * CH
        pltpu.sync_copy(acc.at[pl.ds(off, CH)],
                        aggo.at[pl.ds(c * NACC + off, CH)])


@functools.cache
def _get_sc_agg():
    return pl.kernel(
        _sc_body,
        out_type=jax.ShapeDtypeStruct((NC * NACC, D), jnp.float32),
        mesh=plsc.VectorSubcoreMesh(core_axis_name="c", subcore_axis_name="s",
                                    num_cores=NC, num_subcores=NS),
        compiler_params=pltpu.CompilerParams(needs_layout_passes=False),
        scratch_types=[
            pltpu.VMEM((GCH, CH), jnp.int32),
            pltpu.VMEM((GCH, CH), jnp.int32),
            pltpu.VMEM((GCH, CH), jnp.int32),
            pltpu.VMEM((GCH, CH), jnp.int32),
            pltpu.VMEM((CH, D), jnp.float32),
            pltpu.VMEM((CH, D), jnp.float32),
            pltpu.VMEM((CH, D), jnp.float32),
            pltpu.VMEM((CH, D), jnp.float32),
            pltpu.VMEM_SHARED((NACC, D), jnp.float32),
            pltpu.SemaphoreType.DMA,
            pltpu.SemaphoreType.DMA,
            pltpu.SemaphoreType.DMA,
            pltpu.SemaphoreType.DMA,
        ],
    )


def _scl_body(dsth, nrmh, sclo, dstg, nrmg, sclv):
    c = lax.axis_index("c")
    s = lax.axis_index("s")
    wid = s * NC + c

    zero16 = jnp.zeros((L,), jnp.float32)

    def zscl(i, _):
        sclv[pl.ds(i * L, L)] = zero16
        return 0
    lax.fori_loop(0, NACC // L, zscl, 0)

    def group(g, _):
        row0 = wid * NCHUNK + g * GCH
        pltpu.sync_copy(dsth.at[pl.ds(row0, GCH)], dstg)
        pltpu.sync_copy(nrmh.at[pl.ds(row0, GCH)], nrmg)

        def scat(j, _):
            for k in range(CH // L):
                sl = pl.ds(k * L, L)
                plsc.store_scatter(sclv, [dstg[j, sl]], nrmg[j, sl])
            return 0
        lax.fori_loop(0, GCH, scat, 0)
        return 0
    lax.fori_loop(0, NGRP, group, 0)

    pltpu.sync_copy(sclv, sclo.at[pl.ds(wid * NACC, NACC)])


@functools.cache
def _get_scale():
    return pl.kernel(
        _scl_body,
        out_type=jax.ShapeDtypeStruct((NW * NACC,), jnp.float32),
        mesh=plsc.VectorSubcoreMesh(core_axis_name="c", subcore_axis_name="s",
                                    num_cores=NC, num_subcores=NS),
        compiler_params=pltpu.CompilerParams(needs_layout_passes=False),
        scratch_types=[
            pltpu.VMEM((GCH, CH), jnp.int32),
            pltpu.VMEM((GCH, CH), jnp.float32),
            pltpu.VMEM((NACC,), jnp.float32),
        ],
    )


def _fin_body(bias, scl, agg, xl, o):
    sc = jnp.max(scl[...], axis=0)
    a = agg[0] + agg[1]
    o[...] = jnp.maximum(a * sc[:, None] + xl[0] + bias[...], 0.0)


def _finish(bias2d, scl, agg, xt):
    return pl.pallas_call(
        _fin_body,
        grid=(NACC // BF,),
        in_specs=[
            pl.BlockSpec((1, D), lambda i: (0, 0)),
            pl.BlockSpec((NW, BF), lambda i: (0, i)),
            pl.BlockSpec((NC, BF, D), lambda i: (0, i, 0)),
            pl.BlockSpec((1, BF, D), lambda i: (R, i, 0)),
        ],
        out_specs=pl.BlockSpec((BF, D), lambda i: (i, 0)),
        out_shape=jax.ShapeDtypeStruct((N, D), jnp.float32),
    )(bias2d, scl, agg, xt)


def _layer(x, wcomp, bases, loopw, bias, srcf, etf, dstf, scl):
    xt = _xform(x, wcomp, bases, loopw)
    table = xt.reshape(TBL, D)
    aggf = _get_sc_agg()(table, srcf, etf, dstf)
    agg = aggf.reshape(NC, NACC, D)
    return _finish(bias.reshape(1, D), scl, agg, xt)


def kernel(features, etypes, edge_index, norm, bases1, w_comp1, loop_w1,
           bias1, bases2, w_comp2, loop_w2, bias2):
    src = edge_index[0].astype(jnp.int32)
    dst = edge_index[1].astype(jnp.int32)
    et = etypes.astype(jnp.int32)
    srcf = jnp.concatenate([src, jnp.zeros((PAD,), jnp.int32)])
    srcf = srcf.reshape(NW * NCHUNK, CH)
    etf = jnp.concatenate([et, jnp.zeros((PAD,), jnp.int32)])
    etf = etf.reshape(NW * NCHUNK, CH)
    dstf = jnp.concatenate([dst, jnp.full((PAD,), N, jnp.int32)])
    dstf = dstf.reshape(NW * NCHUNK, CH)
    nrmf = jnp.concatenate([norm[:, 0].astype(jnp.float32),
                            jnp.zeros((PAD,), jnp.float32)])
    nrmf = nrmf.reshape(NW * NCHUNK, CH)
    scl = _get_scale()(dstf, nrmf).reshape(NW, NACC)
    h = _layer(features, w_comp1, bases1, loop_w1, bias1, srcf, etf, dstf, scl)
    h = _layer(h, w_comp2, bases2, loop_w2, bias2, srcf, etf, dstf, scl)
    return h


# speedup vs baseline: 2.7061x; 2.7061x over previous
import functools

import jax
import jax.numpy as jnp
from jax import lax
from jax.experimental import pallas as pl
from jax.experimental.pallas import tpu as pltpu
from jax.experimental.pallas import tpu_sc as plsc

N = 10000
D = 128
R = 8
B = 4
E = 320000

NC, NS, L = 2, 16, 16
NW = NC * NS
CH = 32
NCHUNK = 320
EPW = NCHUNK * CH
EP = NW * EPW
PAD = EP - E
NACC = 10240
RPT = NACC // NS
TBL = N * (R + 1)
BN = 2000
BF = 1024


def _xf_body(wc, bases, lw, x, o):
    xb = x[...]
    cols = []
    for r in range(R):
        w = wc[r, 0] * bases[0]
        for b in range(1, B):
            w = w + wc[r, b] * bases[b]
        cols.append(w)
    cols.append(lw[...])
    wall = jnp.concatenate(cols, axis=1)
    full = jnp.dot(xb, wall, preferred_element_type=jnp.float32)
    for r in range(R + 1):
        o[r] = full[:, r * D:(r + 1) * D]


def _xform(x, wcomp, bases, loopw):
    return pl.pallas_call(
        _xf_body,
        grid=(N // BN,),
        in_specs=[
            pl.BlockSpec(memory_space=pltpu.SMEM),
            pl.BlockSpec((B, D, D), lambda i: (0, 0, 0)),
            pl.BlockSpec((D, D), lambda i: (0, 0)),
            pl.BlockSpec((BN, D), lambda i: (i, 0)),
        ],
        out_specs=pl.BlockSpec((R + 1, BN, D), lambda i: (0, i, 0)),
        out_shape=jax.ShapeDtypeStruct((R + 1, N, D), jnp.float32),
    )(wcomp, bases, loopw, x)


GCH = 32
NGRP = NCHUNK // GCH
NBUF = 4
GRP_C0 = 8
GRP_C1 = 2
CPS = GCH * (GRP_C0 + GRP_C1)


def _sc_body(table, srch, eth, dsth, aggo,
             srcg, etg, gidx, dstg, *bufs):
    rows = bufs[:NBUF]
    acc = bufs[NBUF]
    sems = bufs[NBUF + 1:]
    c = lax.axis_index("c")
    s = lax.axis_index("s")

    zero16 = jnp.zeros((L,), jnp.float32)

    def zrow(i, _):
        for k in range(D // L):
            rows[0][i, pl.ds(k * L, L)] = zero16
        return 0
    lax.fori_loop(0, CH, zrow, 0)
    for m in range(RPT // CH):
        pltpu.sync_copy(rows[0], acc.at[pl.ds(s * RPT + m * CH, CH)])

    plsc.subcore_barrier()

    base = s * CPS + c * (GCH * GRP_C0)
    ngrp = jnp.where(c == 0, GRP_C0, GRP_C1)

    def group(g, _):
        row0 = base + g * GCH
        pltpu.sync_copy(srch.at[pl.ds(row0, GCH)], srcg)
        pltpu.sync_copy(eth.at[pl.ds(row0, GCH)], etg)
        pltpu.sync_copy(dsth.at[pl.ds(row0, GCH)], dstg)

        def gix(j, _):
            for k in range(CH // L):
                sl = pl.ds(k * L, L)
                gidx[j, sl] = etg[j, sl] * N + srcg[j, sl]
            return 0
        lax.fori_loop(0, GCH, gix, 0)

        for b in range(NBUF - 1):
            pltpu.async_copy(table.at[gidx.at[b]], rows[b], sems[b])

        def quad(q, _):
            for b in range(NBUF):
                j = NBUF * q + b
                pltpu.make_async_copy(table.at[gidx.at[j]],
                                      rows[b], sems[b]).wait()

                @pl.when(j + NBUF - 1 < GCH)
                def _():
                    bn = (b + NBUF - 1) % NBUF
                    pltpu.async_copy(table.at[gidx.at[j + NBUF - 1]],
                                     rows[bn], sems[bn])

                pltpu.sync_copy(rows[b], acc.at[dstg.at[j]], add=True)
            return 0
        lax.fori_loop(0, GCH // NBUF, quad, 0)
        return 0
    lax.fori_loop(0, ngrp, group, 0)

    plsc.subcore_barrier()

    for m in range(RPT // CH):
        off = s * RPT + m * CH
        pltpu.sync_copy(acc.at[pl.ds(off, CH)],
                        aggo.at[pl.ds(c * NACC + off, CH)])


@functools.cache
def _get_sc_agg():
    return pl.kernel(
        _sc_body,
        out_type=jax.ShapeDtypeStruct((NC * NACC, D), jnp.float32),
        mesh=plsc.VectorSubcoreMesh(core_axis_name="c", subcore_axis_name="s",
                                    num_cores=NC, num_subcores=NS),
        compiler_params=pltpu.CompilerParams(needs_layout_passes=False),
        scratch_types=(
            [pltpu.VMEM((GCH, CH), jnp.int32)] * 4
            + [pltpu.VMEM((CH, D), jnp.float32)] * NBUF
            + [pltpu.VMEM_SHARED((NACC, D), jnp.float32)]
            + [pltpu.SemaphoreType.DMA] * NBUF
        ),
    )


def _scl_body(dsth, nrmh, sclo, dstg, nrmg, sclv):
    c = lax.axis_index("c")
    s = lax.axis_index("s")
    wid = s * NC + c

    zero16 = jnp.zeros((L,), jnp.float32)

    def zscl(i, _):
        sclv[pl.ds(i * L, L)] = zero16
        return 0
    lax.fori_loop(0, NACC // L, zscl, 0)

    def group(g, _):
        row0 = wid * NCHUNK + g * GCH
        pltpu.sync_copy(dsth.at[pl.ds(row0, GCH)], dstg)
        pltpu.sync_copy(nrmh.at[pl.ds(row0, GCH)], nrmg)

        def scat(j, _):
            for k in range(CH // L):
                sl = pl.ds(k * L, L)
                plsc.store_scatter(sclv, [dstg[j, sl]], nrmg[j, sl])
            return 0
        lax.fori_loop(0, GCH, scat, 0)
        return 0
    lax.fori_loop(0, NGRP, group, 0)

    pltpu.sync_copy(sclv, sclo.at[pl.ds(wid * NACC, NACC)])


@functools.cache
def _get_scale():
    return pl.kernel(
        _scl_body,
        out_type=jax.ShapeDtypeStruct((NW * NACC,), jnp.float32),
        mesh=plsc.VectorSubcoreMesh(core_axis_name="c", subcore_axis_name="s",
                                    num_cores=NC, num_subcores=NS),
        compiler_params=pltpu.CompilerParams(needs_layout_passes=False),
        scratch_types=[
            pltpu.VMEM((GCH, CH), jnp.int32),
            pltpu.VMEM((GCH, CH), jnp.float32),
            pltpu.VMEM((NACC,), jnp.float32),
        ],
    )


def _fin_body(bias, scl, agg, xl, o):
    sc = jnp.max(scl[...], axis=0)
    a = agg[0] + agg[1]
    o[...] = jnp.maximum(a * sc[:, None] + xl[0] + bias[...], 0.0)


def _finish(bias2d, scl, agg, xt):
    return pl.pallas_call(
        _fin_body,
        grid=(NACC // BF,),
        in_specs=[
            pl.BlockSpec((1, D), lambda i: (0, 0)),
            pl.BlockSpec((NW, BF), lambda i: (0, i)),
            pl.BlockSpec((NC, BF, D), lambda i: (0, i, 0)),
            pl.BlockSpec((1, BF, D), lambda i: (R, i, 0)),
        ],
        out_specs=pl.BlockSpec((BF, D), lambda i: (i, 0)),
        out_shape=jax.ShapeDtypeStruct((N, D), jnp.float32),
    )(bias2d, scl, agg, xt)


def _layer(x, wcomp, bases, loopw, bias, srcf, etf, dstf, scl):
    xt = _xform(x, wcomp, bases, loopw)
    table = xt.reshape(TBL, D)
    aggf = _get_sc_agg()(table, srcf, etf, dstf)
    agg = aggf.reshape(NC, NACC, D)
    return _finish(bias.reshape(1, D), scl, agg, xt)


def kernel(features, etypes, edge_index, norm, bases1, w_comp1, loop_w1,
           bias1, bases2, w_comp2, loop_w2, bias2):
    src = edge_index[0].astype(jnp.int32)
    dst = edge_index[1].astype(jnp.int32)
    et = etypes.astype(jnp.int32)
    srcf = jnp.concatenate([src, jnp.zeros((PAD,), jnp.int32)])
    srcf = srcf.reshape(NW * NCHUNK, CH)
    etf = jnp.concatenate([et, jnp.zeros((PAD,), jnp.int32)])
    etf = etf.reshape(NW * NCHUNK, CH)
    dstf = jnp.concatenate([dst, jnp.full((PAD,), N, jnp.int32)])
    dstf = dstf.reshape(NW * NCHUNK, CH)
    nrmf = jnp.concatenate([norm[:, 0].astype(jnp.float32),
                            jnp.zeros((PAD,), jnp.float32)])
    nrmf = nrmf.reshape(NW * NCHUNK, CH)
    scl = _get_scale()(dstf, nrmf).reshape(NW, NACC)
    h = _layer(features, w_comp1, bases1, loop_w1, bias1, srcf, etf, dstf, scl)
    h = _layer(h, w_comp2, bases2, loop_w2, bias2, srcf, etf, dstf, scl)
    return h
